# baseline (speedup 1.0000x reference)
import functools

import jax
import jax.numpy as jnp
from jax import lax
from jax.experimental import pallas as pl
from jax.experimental.pallas import tpu as pltpu
from jax.experimental.pallas import tpu_sc as plsc

N = 10000
E = 320000
IN_C = 128
HID = 256
OUT_C = 128
H = IN_C // 2

NC = 2
NS = 16
NW = NC * NS
K = 128
NCH_DEG = 80
NCH = 160
NBUF = 5
EPAD = NW * NCH_DEG * K
NACC = 10112
ROWS_PT = NACC // NS
_OUT_CHUNKS = [K] * (ROWS_PT // K) + ([ROWS_PT % K] if ROWS_PT % K else [])

R = 1000
GRID = N // R


def _deg_body(dst_hbm, zeros_hbm, ones_hbm, out_hbm, dst_v, zbuf, ones_v, acc_sh):
    cid = lax.axis_index("c")
    sid = lax.axis_index("s")
    wid = cid * NS + sid
    pltpu.sync_copy(dst_hbm.at[wid], dst_v)
    pltpu.sync_copy(zeros_hbm, zbuf)
    pltpu.sync_copy(ones_hbm, ones_v)
    pltpu.sync_copy(zbuf, acc_sh.at[pl.ds(sid * ROWS_PT, ROWS_PT)])
    plsc.subcore_barrier()

    def body(j, carry):
        pltpu.sync_copy(ones_v, acc_sh.at[dst_v.at[j]], add=True)
        return carry

    lax.fori_loop(0, NCH_DEG, body, 0)
    plsc.subcore_barrier()
    pltpu.sync_copy(acc_sh.at[pl.ds(sid * ROWS_PT, ROWS_PT)], zbuf)
    pltpu.sync_copy(zbuf, out_hbm.at[cid, pl.ds(sid * ROWS_PT, ROWS_PT)])


def _agg_body(y_hbm, src_hbm, dst_hbm, zeros_hbm, out_hbm,
              src_v, dst_v, b0, b1, b2, b3, b4, acc_sh,
              g0, g1, g2, g3, g4, s0, s1, s2, s3, s4):
    bufs = [b0, b1, b2, b3, b4]
    gsem = [g0, g1, g2, g3, g4]
    ssem = [s0, s1, s2, s3, s4]
    cid = lax.axis_index("c")
    sid = lax.axis_index("s")
    pltpu.sync_copy(src_hbm.at[cid, sid], src_v)
    pltpu.sync_copy(dst_hbm.at[sid], dst_v)
    pltpu.sync_copy(zeros_hbm, bufs[0])
    base = sid * ROWS_PT
    off = 0
    for sz in _OUT_CHUNKS:
        pltpu.sync_copy(bufs[0].at[pl.ds(0, sz)], acc_sh.at[pl.ds(base + off, sz)])
        off += sz
    plsc.subcore_barrier()

    pltpu.async_copy(y_hbm.at[src_v.at[0]], bufs[0], gsem[0])

    def body(i, carry):
        j0 = 2 * i
        j1 = j0 + 1
        pltpu.async_copy(y_hbm.at[src_v.at[j1]], bufs[1], gsem[1])
        pltpu.make_async_copy(y_hbm.at[src_v.at[j0]], bufs[0], gsem[0]).wait()
        pltpu.sync_copy(bufs[0], acc_sh.at[dst_v.at[j0]], add=True)

        @pl.when(j0 + 2 < NCH)
        def _():
            pltpu.async_copy(y_hbm.at[src_v.at[j0 + 2]], bufs[0], gsem[0])

        pltpu.make_async_copy(y_hbm.at[src_v.at[j1]], bufs[1], gsem[1]).wait()
        pltpu.sync_copy(bufs[1], acc_sh.at[dst_v.at[j1]], add=True)
        return carry

    lax.fori_loop(0, NCH // 2, body, 0)
    plsc.subcore_barrier()
    off = 0
    for sz in _OUT_CHUNKS:
        pltpu.sync_copy(acc_sh.at[pl.ds(base + off, sz)], bufs[0].at[pl.ds(0, sz)])
        pltpu.sync_copy(bufs[0].at[pl.ds(0, sz)], out_hbm.at[cid, pl.ds(base + off, sz)])
        off += sz


@functools.lru_cache(maxsize=None)
def _sc_calls():
    mesh = plsc.VectorSubcoreMesh(core_axis_name="c", subcore_axis_name="s",
                                  num_cores=NC, num_subcores=NS)
    params = pltpu.CompilerParams(use_tc_tiling_on_sc=False)
    deg_call = pl.kernel(
        _deg_body,
        out_type=jax.ShapeDtypeStruct((NC, NACC, 16), jnp.float32),
        mesh=mesh,
        scratch_types=[
            pltpu.VMEM((NCH_DEG, K), jnp.int32),
            pltpu.VMEM((ROWS_PT, 16), jnp.float32),
            pltpu.VMEM((K, 16), jnp.float32),
            pltpu.VMEM_SHARED((NACC, 16), jnp.float32),
        ],
        compiler_params=params,
    )
    agg_call = pl.kernel(
        _agg_body,
        out_type=jax.ShapeDtypeStruct((NC, NACC, H), jnp.float32),
        mesh=mesh,
        scratch_types=(
            [pltpu.VMEM((NCH, K), jnp.int32),
             pltpu.VMEM((NCH, K), jnp.int32)]
            + [pltpu.VMEM((K, H), jnp.float32)] * NBUF
            + [pltpu.VMEM_SHARED((NACC, H), jnp.float32)]
            + [pltpu.SemaphoreType.DMA] * (2 * NBUF)
        ),
        compiler_params=params,
    )
    return deg_call, agg_call


def _dinv(deg_ref):
    return lax.rsqrt(deg_ref[0, :, 0:1] + deg_ref[1, :, 0:1] + 1.0)


def _prep_kernel(deg_ref, x_ref, y_ref):
    dinv = _dinv(deg_ref)
    y_ref[0, :, :] = x_ref[:, 0:H] * dinv
    y_ref[1, :, :] = x_ref[:, H:IN_C] * dinv


def _mm_kernel(z_ref, y_ref, deg_ref, W1_ref, b1_ref, W2_ref, y2_ref):
    dinv = _dinv(deg_ref)
    s0 = (z_ref[0, :, :] + y_ref[0, :, :]) * dinv
    s1 = (z_ref[1, :, :] + y_ref[1, :, :]) * dinv
    h = (jnp.dot(s0, W1_ref[0:H, :], preferred_element_type=jnp.float32)
         + jnp.dot(s1, W1_ref[H:IN_C, :], preferred_element_type=jnp.float32)
         + b1_ref[...])
    h = jnp.maximum(h, 0.0)
    y2_ref[0, :, :] = jnp.dot(h, W2_ref[:, 0:H],
                              preferred_element_type=jnp.float32) * dinv
    y2_ref[1, :, :] = jnp.dot(h, W2_ref[:, H:OUT_C],
                              preferred_element_type=jnp.float32) * dinv


def _fin_kernel(z_ref, y_ref, deg_ref, b2_ref, out_ref):
    dinv = _dinv(deg_ref)
    out_ref[:, 0:H] = (z_ref[0, :, :] + y_ref[0, :, :]) * dinv + b2_ref[:, 0:H]
    out_ref[:, H:OUT_C] = (z_ref[1, :, :] + y_ref[1, :, :]) * dinv + b2_ref[:, H:OUT_C]


def _row_spec(w):
    return pl.BlockSpec((R, w), lambda i: (i, 0))


def _half_spec(w=None):
    return pl.BlockSpec((NC, R, w or H), lambda i: (0, i, 0))


def _full_spec(h, w):
    return pl.BlockSpec((h, w), lambda i: (0, 0))


@functools.lru_cache(maxsize=None)
def _tc_calls():
    prep = pl.pallas_call(
        _prep_kernel,
        grid=(GRID,),
        in_specs=[_half_spec(16), _row_spec(IN_C)],
        out_specs=_half_spec(),
        out_shape=jax.ShapeDtypeStruct((NC, N, H), jnp.float32),
    )
    mm = pl.pallas_call(
        _mm_kernel,
        grid=(GRID,),
        in_specs=[_half_spec(), _half_spec(), _half_spec(16),
                  _full_spec(IN_C, HID), _full_spec(1, HID),
                  _full_spec(HID, OUT_C)],
        out_specs=_half_spec(),
        out_shape=jax.ShapeDtypeStruct((NC, N, H), jnp.float32),
    )
    fin = pl.pallas_call(
        _fin_kernel,
        grid=(GRID,),
        in_specs=[_half_spec(), _half_spec(), _half_spec(16),
                  _full_spec(1, OUT_C)],
        out_specs=_row_spec(OUT_C),
        out_shape=jax.ShapeDtypeStruct((N, OUT_C), jnp.float32),
    )
    return prep, mm, fin


def kernel(x, edge_index, W1, b1, W2, b2):
    deg_call, agg_call = _sc_calls()
    prep, mm, fin = _tc_calls()

    pad = EPAD - E
    src = jnp.concatenate([edge_index[0], jnp.zeros((pad,), jnp.int32)])
    dst = jnp.concatenate([edge_index[1], jnp.full((pad,), N, jnp.int32)])
    dst_deg = dst.reshape(NW, NCH_DEG, K)
    src_agg = jnp.stack([src, src + N]).reshape(NC, NS, NCH, K)
    dst_agg = dst.reshape(NS, NCH, K)

    ones16 = jnp.ones((K, 16), jnp.float32)
    zeros16 = jnp.zeros((ROWS_PT, 16), jnp.float32)
    zerosK = jnp.zeros((K, H), jnp.float32)

    degp = deg_call(dst_deg, zeros16, ones16)

    y1 = prep(degp, x)
    z1 = agg_call(y1.reshape(NC * N, H), src_agg, dst_agg, zerosK)
    y2 = mm(z1, y1, degp, W1, b1.reshape(1, HID), W2)
    z2 = agg_call(y2.reshape(NC * N, H), src_agg, dst_agg, zerosK)
    return fin(z2, y2, degp, b2.reshape(1, OUT_C))


# speedup vs baseline: 1.5340x; 1.5340x over previous
import functools

import jax
import jax.numpy as jnp
from jax import lax
from jax.experimental import pallas as pl
from jax.experimental.pallas import tpu as pltpu
from jax.experimental.pallas import tpu_sc as plsc

N = 10000
E = 320000
IN_C = 128
HID = 256
OUT_C = 128
H = IN_C // 2

NC = 2
NS = 16
NW = NC * NS
K = 112
NCH_DEG = 90
NCH = 180
EPAD = NW * NCH_DEG * K
NACC = 10112
ROWS_PT = NACC // NS
_OUT_CHUNKS = [K] * (ROWS_PT // K) + ([ROWS_PT % K] if ROWS_PT % K else [])

R = 1000
GRID = N // R


def _deg_body(dst_hbm, zeros_hbm, ones_hbm, out_hbm, dst_v, zbuf, ones_v, acc_sh):
    cid = lax.axis_index("c")
    sid = lax.axis_index("s")
    wid = cid * NS + sid
    pltpu.sync_copy(dst_hbm.at[wid], dst_v)
    pltpu.sync_copy(zeros_hbm, zbuf)
    pltpu.sync_copy(ones_hbm, ones_v)
    pltpu.sync_copy(zbuf, acc_sh.at[pl.ds(sid * ROWS_PT, ROWS_PT)])
    plsc.subcore_barrier()

    def body(j, carry):
        pltpu.sync_copy(ones_v, acc_sh.at[dst_v.at[j]], add=True)
        return carry

    lax.fori_loop(0, NCH_DEG, body, 0)
    plsc.subcore_barrier()
    pltpu.sync_copy(acc_sh.at[pl.ds(sid * ROWS_PT, ROWS_PT)], zbuf)
    pltpu.sync_copy(zbuf, out_hbm.at[cid, pl.ds(sid * ROWS_PT, ROWS_PT)])


def _agg_body(y_hbm, src_hbm, dst_hbm, zeros_hbm, out_hbm,
              src_v, dst_v, b0, b1, acc_sh, g0, g1):
    bufs = [b0, b1]
    gsem = [g0, g1]
    cid = lax.axis_index("c")
    sid = lax.axis_index("s")
    pltpu.sync_copy(src_hbm.at[cid, sid], src_v)
    pltpu.sync_copy(dst_hbm.at[sid], dst_v)
    pltpu.sync_copy(zeros_hbm, bufs[0])
    base = sid * ROWS_PT
    off = 0
    for sz in _OUT_CHUNKS:
        pltpu.sync_copy(bufs[0].at[pl.ds(0, sz)], acc_sh.at[pl.ds(base + off, sz)])
        off += sz
    plsc.subcore_barrier()

    pltpu.async_copy(y_hbm.at[src_v.at[0]], bufs[0], gsem[0])

    def body(i, carry):
        j0 = 2 * i
        j1 = j0 + 1
        pltpu.async_copy(y_hbm.at[src_v.at[j1]], bufs[1], gsem[1])
        pltpu.make_async_copy(y_hbm.at[src_v.at[j0]], bufs[0], gsem[0]).wait()
        pltpu.sync_copy(bufs[0], acc_sh.at[dst_v.at[j0]], add=True)

        @pl.when(j0 + 2 < NCH)
        def _():
            pltpu.async_copy(y_hbm.at[src_v.at[j0 + 2]], bufs[0], gsem[0])

        pltpu.make_async_copy(y_hbm.at[src_v.at[j1]], bufs[1], gsem[1]).wait()
        pltpu.sync_copy(bufs[1], acc_sh.at[dst_v.at[j1]], add=True)
        return carry

    lax.fori_loop(0, NCH // 2, body, 0)
    plsc.subcore_barrier()
    off = 0
    for sz in _OUT_CHUNKS:
        pltpu.sync_copy(acc_sh.at[pl.ds(base + off, sz)], bufs[0].at[pl.ds(0, sz)])
        pltpu.sync_copy(bufs[0].at[pl.ds(0, sz)], out_hbm.at[cid, pl.ds(base + off, sz)])
        off += sz


@functools.lru_cache(maxsize=None)
def _sc_calls():
    mesh = plsc.VectorSubcoreMesh(core_axis_name="c", subcore_axis_name="s",
                                  num_cores=NC, num_subcores=NS)
    params = pltpu.CompilerParams(use_tc_tiling_on_sc=False)
    deg_call = pl.kernel(
        _deg_body,
        out_type=jax.ShapeDtypeStruct((NC, NACC, 16), jnp.float32),
        mesh=mesh,
        scratch_types=[
            pltpu.VMEM((NCH_DEG, K), jnp.int32),
            pltpu.VMEM((ROWS_PT, 16), jnp.float32),
            pltpu.VMEM((K, 16), jnp.float32),
            pltpu.VMEM_SHARED((NACC, 16), jnp.float32),
        ],
        compiler_params=params,
    )
    agg_call = pl.kernel(
        _agg_body,
        out_type=jax.ShapeDtypeStruct((NC, NACC, H), jnp.float32),
        mesh=mesh,
        scratch_types=(
            [pltpu.VMEM((NCH, K), jnp.int32),
             pltpu.VMEM((NCH, K), jnp.int32)]
            + [pltpu.VMEM((K, H), jnp.float32)] * 2
            + [pltpu.VMEM_SHARED((NACC, H), jnp.float32)]
            + [pltpu.SemaphoreType.DMA] * 2
        ),
        compiler_params=params,
    )
    return deg_call, agg_call


def _dinv(deg_ref):
    return lax.rsqrt(deg_ref[0, :, 0:1] + deg_ref[1, :, 0:1] + 1.0)


def _prep_kernel(deg_ref, x_ref, y_ref):
    dinv = _dinv(deg_ref)
    y_ref[0, :, :] = x_ref[:, 0:H] * dinv
    y_ref[1, :, :] = x_ref[:, H:IN_C] * dinv


def _mm_kernel(z_ref, y_ref, deg_ref, W1_ref, b1_ref, W2_ref, y2_ref):
    dinv = _dinv(deg_ref)
    s0 = (z_ref[0, :, :] + y_ref[0, :, :]) * dinv
    s1 = (z_ref[1, :, :] + y_ref[1, :, :]) * dinv
    h = (jnp.dot(s0, W1_ref[0:H, :], preferred_element_type=jnp.float32)
         + jnp.dot(s1, W1_ref[H:IN_C, :], preferred_element_type=jnp.float32)
         + b1_ref[...])
    h = jnp.maximum(h, 0.0)
    y2_ref[0, :, :] = jnp.dot(h, W2_ref[:, 0:H],
                              preferred_element_type=jnp.float32) * dinv
    y2_ref[1, :, :] = jnp.dot(h, W2_ref[:, H:OUT_C],
                              preferred_element_type=jnp.float32) * dinv


def _fin_kernel(z_ref, y_ref, deg_ref, b2_ref, out_ref):
    dinv = _dinv(deg_ref)
    out_ref[:, 0:H] = (z_ref[0, :, :] + y_ref[0, :, :]) * dinv + b2_ref[:, 0:H]
    out_ref[:, H:OUT_C] = (z_ref[1, :, :] + y_ref[1, :, :]) * dinv + b2_ref[:, H:OUT_C]


def _row_spec(w):
    return pl.BlockSpec((R, w), lambda i: (i, 0))


def _half_spec(w=None):
    return pl.BlockSpec((NC, R, w or H), lambda i: (0, i, 0))


def _full_spec(h, w):
    return pl.BlockSpec((h, w), lambda i: (0, 0))


@functools.lru_cache(maxsize=None)
def _tc_calls():
    prep = pl.pallas_call(
        _prep_kernel,
        grid=(GRID,),
        in_specs=[_half_spec(16), _row_spec(IN_C)],
        out_specs=_half_spec(),
        out_shape=jax.ShapeDtypeStruct((NC, N, H), jnp.float32),
    )
    mm = pl.pallas_call(
        _mm_kernel,
        grid=(GRID,),
        in_specs=[_half_spec(), _half_spec(), _half_spec(16),
                  _full_spec(IN_C, HID), _full_spec(1, HID),
                  _full_spec(HID, OUT_C)],
        out_specs=_half_spec(),
        out_shape=jax.ShapeDtypeStruct((NC, N, H), jnp.float32),
    )
    fin = pl.pallas_call(
        _fin_kernel,
        grid=(GRID,),
        in_specs=[_half_spec(), _half_spec(), _half_spec(16),
                  _full_spec(1, OUT_C)],
        out_specs=_row_spec(OUT_C),
        out_shape=jax.ShapeDtypeStruct((N, OUT_C), jnp.float32),
    )
    return prep, mm, fin


def kernel(x, edge_index, W1, b1, W2, b2):
    deg_call, agg_call = _sc_calls()
    prep, mm, fin = _tc_calls()

    pad = EPAD - E
    src = jnp.concatenate([edge_index[0], jnp.zeros((pad,), jnp.int32)])
    dst = jnp.concatenate([edge_index[1], jnp.full((pad,), N, jnp.int32)])
    dst_deg = dst.reshape(NW, NCH_DEG, K)
    src_agg = jnp.stack([src, src + N]).reshape(NC, NS, NCH, K)
    dst_agg = dst.reshape(NS, NCH, K)

    ones16 = jnp.ones((K, 16), jnp.float32)
    zeros16 = jnp.zeros((ROWS_PT, 16), jnp.float32)
    zerosK = jnp.zeros((K, H), jnp.float32)

    degp = deg_call(dst_deg, zeros16, ones16)

    y1 = prep(degp, x)
    z1 = agg_call(y1.reshape(NC * N, H), src_agg, dst_agg, zerosK)
    y2 = mm(z1, y1, degp, W1, b1.reshape(1, HID), W2)
    z2 = agg_call(y2.reshape(NC * N, H), src_agg, dst_agg, zerosK)
    return fin(z2, y2, degp, b2.reshape(1, OUT_C))


# speedup vs baseline: 1.7299x; 1.1277x over previous
import functools

import jax
import jax.numpy as jnp
from jax import lax
from jax.experimental import pallas as pl
from jax.experimental.pallas import tpu as pltpu
from jax.experimental.pallas import tpu_sc as plsc

N = 10000
E = 320000
IN_C = 128
HID = 256
OUT_C = 128
H = IN_C // 2

NC = 2
NS = 16
NW = NC * NS
K = 112
NCH_DEG = 90
NCH = 180
EPAD = NW * NCH_DEG * K
NACC = 10112
ROWS_PT = NACC // NS
_OUT_CHUNKS = [K] * (ROWS_PT // K) + ([ROWS_PT % K] if ROWS_PT % K else [])

R = 1000
GRID = N // R


def _deg_body(dst_hbm, zeros_hbm, ones_hbm, out_hbm, dst_v, zbuf, ones_v, acc_sh):
    cid = lax.axis_index("c")
    sid = lax.axis_index("s")
    wid = cid * NS + sid
    pltpu.sync_copy(dst_hbm.at[wid], dst_v)
    pltpu.sync_copy(zeros_hbm, zbuf)
    pltpu.sync_copy(ones_hbm, ones_v)
    pltpu.sync_copy(zbuf, acc_sh.at[pl.ds(sid * ROWS_PT, ROWS_PT)])
    plsc.subcore_barrier()

    def body(j, carry):
        pltpu.sync_copy(ones_v, acc_sh.at[dst_v.at[j]], add=True)
        return carry

    lax.fori_loop(0, NCH_DEG, body, 0)
    plsc.subcore_barrier()
    pltpu.sync_copy(acc_sh.at[pl.ds(sid * ROWS_PT, ROWS_PT)], zbuf)
    pltpu.sync_copy(zbuf, out_hbm.at[cid, pl.ds(sid * ROWS_PT, ROWS_PT)])


NBUF = 6


def _agg_body(y_hbm, src_hbm, dst_hbm, zeros_hbm, out_hbm,
              src_v, dst_v, b0, b1, b2, b3, b4, b5, acc_sh,
              g0, g1, g2, g3, g4, g5, s0, s1, s2, s3, s4, s5):
    bufs = [b0, b1, b2, b3, b4, b5]
    gsem = [g0, g1, g2, g3, g4, g5]
    ssem = [s0, s1, s2, s3, s4, s5]
    cid = lax.axis_index("c")
    sid = lax.axis_index("s")
    pltpu.sync_copy(src_hbm.at[cid, sid], src_v)
    pltpu.sync_copy(dst_hbm.at[sid], dst_v)
    pltpu.sync_copy(zeros_hbm, bufs[0])
    base = sid * ROWS_PT
    off = 0
    for sz in _OUT_CHUNKS:
        pltpu.sync_copy(bufs[0].at[pl.ds(0, sz)], acc_sh.at[pl.ds(base + off, sz)])
        off += sz
    plsc.subcore_barrier()

    for b in range(NBUF):
        pltpu.async_copy(y_hbm.at[src_v.at[b]], bufs[b], gsem[b])

    def body(i, carry):
        for b in range(NBUF):
            j = NBUF * i + b
            pltpu.make_async_copy(y_hbm.at[src_v.at[j]], bufs[b], gsem[b]).wait()
            pltpu.async_copy(bufs[b], acc_sh.at[dst_v.at[j]], ssem[b], add=True)
            bp = (b - 1) % NBUF
            jn = j + NBUF - 1

            @pl.when(jnp.logical_and(jn >= NBUF, jn < NCH))
            def _():
                pltpu.make_async_copy(
                    bufs[bp], acc_sh.at[dst_v.at[j - 1]], ssem[bp]).wait()
                pltpu.async_copy(y_hbm.at[src_v.at[jn]], bufs[bp], gsem[bp])

        return carry

    lax.fori_loop(0, NCH // NBUF, body, 0)
    for b in range(NBUF):
        pltpu.make_async_copy(
            bufs[b], acc_sh.at[dst_v.at[NCH - NBUF + b]], ssem[b]).wait()
    plsc.subcore_barrier()
    off = 0
    for sz in _OUT_CHUNKS:
        pltpu.sync_copy(acc_sh.at[pl.ds(base + off, sz)], bufs[0].at[pl.ds(0, sz)])
        pltpu.sync_copy(bufs[0].at[pl.ds(0, sz)], out_hbm.at[cid, pl.ds(base + off, sz)])
        off += sz


@functools.lru_cache(maxsize=None)
def _sc_calls():
    mesh = plsc.VectorSubcoreMesh(core_axis_name="c", subcore_axis_name="s",
                                  num_cores=NC, num_subcores=NS)
    params = pltpu.CompilerParams(use_tc_tiling_on_sc=False)
    deg_call = pl.kernel(
        _deg_body,
        out_type=jax.ShapeDtypeStruct((NC, NACC, 16), jnp.float32),
        mesh=mesh,
        scratch_types=[
            pltpu.VMEM((NCH_DEG, K), jnp.int32),
            pltpu.VMEM((ROWS_PT, 16), jnp.float32),
            pltpu.VMEM((K, 16), jnp.float32),
            pltpu.VMEM_SHARED((NACC, 16), jnp.float32),
        ],
        compiler_params=params,
    )
    agg_call = pl.kernel(
        _agg_body,
        out_type=jax.ShapeDtypeStruct((NC, NACC, H), jnp.float32),
        mesh=mesh,
        scratch_types=(
            [pltpu.VMEM((NCH, K), jnp.int32),
             pltpu.VMEM((NCH, K), jnp.int32)]
            + [pltpu.VMEM((K, H), jnp.float32)] * NBUF
            + [pltpu.VMEM_SHARED((NACC, H), jnp.float32)]
            + [pltpu.SemaphoreType.DMA] * (2 * NBUF)
        ),
        compiler_params=params,
    )
    return deg_call, agg_call


def _dinv(deg_ref):
    return lax.rsqrt(deg_ref[0, :, 0:1] + deg_ref[1, :, 0:1] + 1.0)


def _prep_kernel(deg_ref, x_ref, y_ref):
    dinv = _dinv(deg_ref)
    y_ref[0, :, :] = x_ref[:, 0:H] * dinv
    y_ref[1, :, :] = x_ref[:, H:IN_C] * dinv


def _mm_kernel(z_ref, y_ref, deg_ref, W1_ref, b1_ref, W2_ref, y2_ref):
    dinv = _dinv(deg_ref)
    s0 = (z_ref[0, :, :] + y_ref[0, :, :]) * dinv
    s1 = (z_ref[1, :, :] + y_ref[1, :, :]) * dinv
    h = (jnp.dot(s0, W1_ref[0:H, :], preferred_element_type=jnp.float32)
         + jnp.dot(s1, W1_ref[H:IN_C, :], preferred_element_type=jnp.float32)
         + b1_ref[...])
    h = jnp.maximum(h, 0.0)
    y2_ref[0, :, :] = jnp.dot(h, W2_ref[:, 0:H],
                              preferred_element_type=jnp.float32) * dinv
    y2_ref[1, :, :] = jnp.dot(h, W2_ref[:, H:OUT_C],
                              preferred_element_type=jnp.float32) * dinv


def _fin_kernel(z_ref, y_ref, deg_ref, b2_ref, out_ref):
    dinv = _dinv(deg_ref)
    out_ref[:, 0:H] = (z_ref[0, :, :] + y_ref[0, :, :]) * dinv + b2_ref[:, 0:H]
    out_ref[:, H:OUT_C] = (z_ref[1, :, :] + y_ref[1, :, :]) * dinv + b2_ref[:, H:OUT_C]


def _row_spec(w):
    return pl.BlockSpec((R, w), lambda i: (i, 0))


def _half_spec(w=None):
    return pl.BlockSpec((NC, R, w or H), lambda i: (0, i, 0))


def _full_spec(h, w):
    return pl.BlockSpec((h, w), lambda i: (0, 0))


@functools.lru_cache(maxsize=None)
def _tc_calls():
    prep = pl.pallas_call(
        _prep_kernel,
        grid=(GRID,),
        in_specs=[_half_spec(16), _row_spec(IN_C)],
        out_specs=_half_spec(),
        out_shape=jax.ShapeDtypeStruct((NC, N, H), jnp.float32),
    )
    mm = pl.pallas_call(
        _mm_kernel,
        grid=(GRID,),
        in_specs=[_half_spec(), _half_spec(), _half_spec(16),
                  _full_spec(IN_C, HID), _full_spec(1, HID),
                  _full_spec(HID, OUT_C)],
        out_specs=_half_spec(),
        out_shape=jax.ShapeDtypeStruct((NC, N, H), jnp.float32),
    )
    fin = pl.pallas_call(
        _fin_kernel,
        grid=(GRID,),
        in_specs=[_half_spec(), _half_spec(), _half_spec(16),
                  _full_spec(1, OUT_C)],
        out_specs=_row_spec(OUT_C),
        out_shape=jax.ShapeDtypeStruct((N, OUT_C), jnp.float32),
    )
    return prep, mm, fin


def kernel(x, edge_index, W1, b1, W2, b2):
    deg_call, agg_call = _sc_calls()
    prep, mm, fin = _tc_calls()

    pad = EPAD - E
    src = jnp.concatenate([edge_index[0], jnp.zeros((pad,), jnp.int32)])
    dst = jnp.concatenate([edge_index[1], jnp.full((pad,), N, jnp.int32)])
    dst_deg = dst.reshape(NW, NCH_DEG, K)
    src_agg = jnp.stack([src, src + N]).reshape(NC, NS, NCH, K)
    dst_agg = dst.reshape(NS, NCH, K)

    ones16 = jnp.ones((K, 16), jnp.float32)
    zeros16 = jnp.zeros((ROWS_PT, 16), jnp.float32)
    zerosK = jnp.zeros((K, H), jnp.float32)

    degp = deg_call(dst_deg, zeros16, ones16)

    y1 = prep(degp, x)
    z1 = agg_call(y1.reshape(NC * N, H), src_agg, dst_agg, zerosK)
    y2 = mm(z1, y1, degp, W1, b1.reshape(1, HID), W2)
    z2 = agg_call(y2.reshape(NC * N, H), src_agg, dst_agg, zerosK)
    return fin(z2, y2, degp, b2.reshape(1, OUT_C))


# speedup vs baseline: 1.7409x; 1.0064x over previous
import functools

import jax
import jax.numpy as jnp
from jax import lax
from jax.experimental import pallas as pl
from jax.experimental.pallas import tpu as pltpu
from jax.experimental.pallas import tpu_sc as plsc

N = 10000
E = 320000
IN_C = 128
HID = 256
OUT_C = 128
H = IN_C // 2

NC = 2
NS = 16
NW = NC * NS
K = 80
NCH_DEG = 126
NCH = 252
EPAD = NW * NCH_DEG * K
NACC = 10112
ROWS_PT = NACC // NS
_OUT_CHUNKS = [K] * (ROWS_PT // K) + ([ROWS_PT % K] if ROWS_PT % K else [])

R = 1000
GRID = N // R


def _deg_body(dst_hbm, zeros_hbm, ones_hbm, out_hbm, dst_v, zbuf, ones_v, acc_sh):
    cid = lax.axis_index("c")
    sid = lax.axis_index("s")
    wid = cid * NS + sid
    pltpu.sync_copy(dst_hbm.at[wid], dst_v)
    pltpu.sync_copy(zeros_hbm, zbuf)
    pltpu.sync_copy(ones_hbm, ones_v)
    pltpu.sync_copy(zbuf, acc_sh.at[pl.ds(sid * ROWS_PT, ROWS_PT)])
    plsc.subcore_barrier()

    def body(j, carry):
        pltpu.sync_copy(ones_v, acc_sh.at[dst_v.at[j]], add=True)
        return carry

    lax.fori_loop(0, NCH_DEG, body, 0)
    plsc.subcore_barrier()
    pltpu.sync_copy(acc_sh.at[pl.ds(sid * ROWS_PT, ROWS_PT)], zbuf)
    pltpu.sync_copy(zbuf, out_hbm.at[cid, pl.ds(sid * ROWS_PT, ROWS_PT)])


NBUF = 9


def _agg_body(y_hbm, src_hbm, dst_hbm, zeros_hbm, out_hbm,
              src_v, dst_v, b0, b1, b2, b3, b4, b5, b6, b7, b8, acc_sh,
              g0, g1, g2, g3, g4, g5, g6, g7, g8,
              s0, s1, s2, s3, s4, s5, s6, s7, s8):
    bufs = [b0, b1, b2, b3, b4, b5, b6, b7, b8]
    gsem = [g0, g1, g2, g3, g4, g5, g6, g7, g8]
    ssem = [s0, s1, s2, s3, s4, s5, s6, s7, s8]
    cid = lax.axis_index("c")
    sid = lax.axis_index("s")
    pltpu.sync_copy(src_hbm.at[cid, sid], src_v)
    pltpu.sync_copy(dst_hbm.at[sid], dst_v)
    pltpu.sync_copy(zeros_hbm, bufs[0])
    base = sid * ROWS_PT
    off = 0
    for sz in _OUT_CHUNKS:
        pltpu.sync_copy(bufs[0].at[pl.ds(0, sz)], acc_sh.at[pl.ds(base + off, sz)])
        off += sz
    plsc.subcore_barrier()

    for b in range(NBUF):
        pltpu.async_copy(y_hbm.at[src_v.at[b]], bufs[b], gsem[b])

    def body(i, carry):
        for b in range(NBUF):
            j = NBUF * i + b
            pltpu.make_async_copy(y_hbm.at[src_v.at[j]], bufs[b], gsem[b]).wait()
            pltpu.async_copy(bufs[b], acc_sh.at[dst_v.at[j]], ssem[b], add=True)
            bp = (b - 1) % NBUF
            jn = j + NBUF - 1

            @pl.when(jnp.logical_and(jn >= NBUF, jn < NCH))
            def _():
                pltpu.make_async_copy(
                    bufs[bp], acc_sh.at[dst_v.at[j - 1]], ssem[bp]).wait()
                pltpu.async_copy(y_hbm.at[src_v.at[jn]], bufs[bp], gsem[bp])

        return carry

    lax.fori_loop(0, NCH // NBUF, body, 0)
    for b in range(NBUF):
        pltpu.make_async_copy(
            bufs[b], acc_sh.at[dst_v.at[NCH - NBUF + b]], ssem[b]).wait()
    plsc.subcore_barrier()
    off = 0
    for sz in _OUT_CHUNKS:
        pltpu.sync_copy(acc_sh.at[pl.ds(base + off, sz)], bufs[0].at[pl.ds(0, sz)])
        pltpu.sync_copy(bufs[0].at[pl.ds(0, sz)], out_hbm.at[cid, pl.ds(base + off, sz)])
        off += sz


@functools.lru_cache(maxsize=None)
def _sc_calls():
    mesh = plsc.VectorSubcoreMesh(core_axis_name="c", subcore_axis_name="s",
                                  num_cores=NC, num_subcores=NS)
    params = pltpu.CompilerParams(use_tc_tiling_on_sc=False)
    deg_call = pl.kernel(
        _deg_body,
        out_type=jax.ShapeDtypeStruct((NC, NACC, 16), jnp.float32),
        mesh=mesh,
        scratch_types=[
            pltpu.VMEM((NCH_DEG, K), jnp.int32),
            pltpu.VMEM((ROWS_PT, 16), jnp.float32),
            pltpu.VMEM((K, 16), jnp.float32),
            pltpu.VMEM_SHARED((NACC, 16), jnp.float32),
        ],
        compiler_params=params,
    )
    agg_call = pl.kernel(
        _agg_body,
        out_type=jax.ShapeDtypeStruct((NC, NACC, H), jnp.float32),
        mesh=mesh,
        scratch_types=(
            [pltpu.VMEM((NCH, K), jnp.int32),
             pltpu.VMEM((NCH, K), jnp.int32)]
            + [pltpu.VMEM((K, H), jnp.float32)] * NBUF
            + [pltpu.VMEM_SHARED((NACC, H), jnp.float32)]
            + [pltpu.SemaphoreType.DMA] * (2 * NBUF)
        ),
        compiler_params=params,
    )
    return deg_call, agg_call


def _dinv(deg_ref):
    return lax.rsqrt(deg_ref[0, :, 0:1] + deg_ref[1, :, 0:1] + 1.0)


def _prep_kernel(deg_ref, x_ref, y_ref):
    dinv = _dinv(deg_ref)
    y_ref[0, :, :] = x_ref[:, 0:H] * dinv
    y_ref[1, :, :] = x_ref[:, H:IN_C] * dinv


def _mm_kernel(z_ref, y_ref, deg_ref, W1_ref, b1_ref, W2_ref, y2_ref):
    dinv = _dinv(deg_ref)
    s0 = (z_ref[0, :, :] + y_ref[0, :, :]) * dinv
    s1 = (z_ref[1, :, :] + y_ref[1, :, :]) * dinv
    h = (jnp.dot(s0, W1_ref[0:H, :], preferred_element_type=jnp.float32)
         + jnp.dot(s1, W1_ref[H:IN_C, :], preferred_element_type=jnp.float32)
         + b1_ref[...])
    h = jnp.maximum(h, 0.0)
    y2_ref[0, :, :] = jnp.dot(h, W2_ref[:, 0:H],
                              preferred_element_type=jnp.float32) * dinv
    y2_ref[1, :, :] = jnp.dot(h, W2_ref[:, H:OUT_C],
                              preferred_element_type=jnp.float32) * dinv


def _fin_kernel(z_ref, y_ref, deg_ref, b2_ref, out_ref):
    dinv = _dinv(deg_ref)
    out_ref[:, 0:H] = (z_ref[0, :, :] + y_ref[0, :, :]) * dinv + b2_ref[:, 0:H]
    out_ref[:, H:OUT_C] = (z_ref[1, :, :] + y_ref[1, :, :]) * dinv + b2_ref[:, H:OUT_C]


def _row_spec(w):
    return pl.BlockSpec((R, w), lambda i: (i, 0))


def _half_spec(w=None):
    return pl.BlockSpec((NC, R, w or H), lambda i: (0, i, 0))


def _full_spec(h, w):
    return pl.BlockSpec((h, w), lambda i: (0, 0))


@functools.lru_cache(maxsize=None)
def _tc_calls():
    prep = pl.pallas_call(
        _prep_kernel,
        grid=(GRID,),
        in_specs=[_half_spec(16), _row_spec(IN_C)],
        out_specs=_half_spec(),
        out_shape=jax.ShapeDtypeStruct((NC, N, H), jnp.float32),
    )
    mm = pl.pallas_call(
        _mm_kernel,
        grid=(GRID,),
        in_specs=[_half_spec(), _half_spec(), _half_spec(16),
                  _full_spec(IN_C, HID), _full_spec(1, HID),
                  _full_spec(HID, OUT_C)],
        out_specs=_half_spec(),
        out_shape=jax.ShapeDtypeStruct((NC, N, H), jnp.float32),
    )
    fin = pl.pallas_call(
        _fin_kernel,
        grid=(GRID,),
        in_specs=[_half_spec(), _half_spec(), _half_spec(16),
                  _full_spec(1, OUT_C)],
        out_specs=_row_spec(OUT_C),
        out_shape=jax.ShapeDtypeStruct((N, OUT_C), jnp.float32),
    )
    return prep, mm, fin


def kernel(x, edge_index, W1, b1, W2, b2):
    deg_call, agg_call = _sc_calls()
    prep, mm, fin = _tc_calls()

    pad = EPAD - E
    src = jnp.concatenate([edge_index[0], jnp.zeros((pad,), jnp.int32)])
    dst = jnp.concatenate([edge_index[1], jnp.full((pad,), N, jnp.int32)])
    dst_deg = dst.reshape(NW, NCH_DEG, K)
    src_agg = jnp.stack([src, src + N]).reshape(NC, NS, NCH, K)
    dst_agg = dst.reshape(NS, NCH, K)

    ones16 = jnp.ones((K, 16), jnp.float32)
    zeros16 = jnp.zeros((ROWS_PT, 16), jnp.float32)
    zerosK = jnp.zeros((K, H), jnp.float32)

    degp = deg_call(dst_deg, zeros16, ones16)

    y1 = prep(degp, x)
    z1 = agg_call(y1.reshape(NC * N, H), src_agg, dst_agg, zerosK)
    y2 = mm(z1, y1, degp, W1, b1.reshape(1, HID), W2)
    z2 = agg_call(y2.reshape(NC * N, H), src_agg, dst_agg, zerosK)
    return fin(z2, y2, degp, b2.reshape(1, OUT_C))


# speedup vs baseline: 2.4139x; 1.3866x over previous
import functools

import jax
import jax.numpy as jnp
from jax import lax
from jax.experimental import pallas as pl
from jax.experimental.pallas import tpu as pltpu
from jax.experimental.pallas import tpu_sc as plsc

N = 10000
E = 320000
IN_C = 128
HID = 256
OUT_C = 128
H = IN_C // 2

NC = 2
NS = 16
NW = NC * NS
K = 80
NCH_DEG = 125
NCH = 250
DEG_WIN = 8
EPAD = NW * NCH_DEG * K
NACC = 10112
ROWS_PT = NACC // NS
_OUT_CHUNKS = [K] * (ROWS_PT // K) + ([ROWS_PT % K] if ROWS_PT % K else [])

R = 1000
GRID = N // R


def _deg_body(dst_hbm, zeros_hbm, ones_hbm, out_hbm, dst_v, zbuf, ones_v, acc_sh,
              ssem):
    cid = lax.axis_index("c")
    sid = lax.axis_index("s")
    wid = cid * NS + sid
    pltpu.sync_copy(dst_hbm.at[wid], dst_v)
    pltpu.sync_copy(zeros_hbm, zbuf)
    pltpu.sync_copy(ones_hbm, ones_v)
    pltpu.sync_copy(zbuf, acc_sh.at[pl.ds(sid * ROWS_PT, ROWS_PT)])
    plsc.subcore_barrier()

    def body(j, carry):
        pltpu.async_copy(ones_v, acc_sh.at[dst_v.at[j]], ssem, add=True)

        @pl.when(j >= DEG_WIN)
        def _():
            pltpu.make_async_copy(ones_v, acc_sh.at[dst_v.at[0]], ssem).wait()

        return carry

    lax.fori_loop(0, NCH_DEG, body, 0)
    for _ in range(DEG_WIN):
        pltpu.make_async_copy(ones_v, acc_sh.at[dst_v.at[0]], ssem).wait()
    plsc.subcore_barrier()
    pltpu.sync_copy(acc_sh.at[pl.ds(sid * ROWS_PT, ROWS_PT)], zbuf)
    pltpu.sync_copy(zbuf, out_hbm.at[cid, pl.ds(sid * ROWS_PT, ROWS_PT)])


NBUF = 5


def _agg_body(y_hbm, src_hbm, dst_hbm, zeros_hbm, out_hbm,
              src_v, dst_v, b0, b1, b2, b3, b4, acc_sh,
              g0, g1, g2, g3, g4, s0, s1, s2, s3, s4):
    bufs = [b0, b1, b2, b3, b4]
    gsem = [g0, g1, g2, g3, g4]
    ssem = [s0, s1, s2, s3, s4]
    cid = lax.axis_index("c")
    sid = lax.axis_index("s")
    pltpu.sync_copy(src_hbm.at[cid, sid], src_v)
    pltpu.sync_copy(dst_hbm.at[sid], dst_v)
    pltpu.sync_copy(zeros_hbm, bufs[0])
    base = sid * ROWS_PT
    off = 0
    for sz in _OUT_CHUNKS:
        pltpu.sync_copy(bufs[0].at[pl.ds(0, sz)], acc_sh.at[pl.ds(base + off, sz)])
        off += sz
    plsc.subcore_barrier()

    for b in range(NBUF):
        pltpu.async_copy(y_hbm.at[src_v.at[b]], bufs[b], gsem[b])

    def body(i, carry):
        for b in range(NBUF):
            j = NBUF * i + b
            pltpu.make_async_copy(y_hbm.at[src_v.at[j]], bufs[b], gsem[b]).wait()
            pltpu.async_copy(bufs[b], acc_sh.at[dst_v.at[j]], ssem[b], add=True)
            bp = (b - 1) % NBUF
            jn = j + NBUF - 1

            @pl.when(jnp.logical_and(jn >= NBUF, jn < NCH))
            def _():
                pltpu.make_async_copy(
                    bufs[bp], acc_sh.at[dst_v.at[j - 1]], ssem[bp]).wait()
                pltpu.async_copy(y_hbm.at[src_v.at[jn]], bufs[bp], gsem[bp])

        return carry

    lax.fori_loop(0, NCH // NBUF, body, 0)
    for b in range(NBUF):
        pltpu.make_async_copy(
            bufs[b], acc_sh.at[dst_v.at[NCH - NBUF + b]], ssem[b]).wait()
    plsc.subcore_barrier()
    off = 0
    for sz in _OUT_CHUNKS:
        pltpu.sync_copy(acc_sh.at[pl.ds(base + off, sz)], bufs[0].at[pl.ds(0, sz)])
        pltpu.sync_copy(bufs[0].at[pl.ds(0, sz)], out_hbm.at[cid, pl.ds(base + off, sz)])
        off += sz


@functools.lru_cache(maxsize=None)
def _sc_calls():
    mesh = plsc.VectorSubcoreMesh(core_axis_name="c", subcore_axis_name="s",
                                  num_cores=NC, num_subcores=NS)
    params = pltpu.CompilerParams(use_tc_tiling_on_sc=False)
    deg_call = pl.kernel(
        _deg_body,
        out_type=jax.ShapeDtypeStruct((NC, NACC, 16), jnp.float32),
        mesh=mesh,
        scratch_types=[
            pltpu.VMEM((NCH_DEG, K), jnp.int32),
            pltpu.VMEM((ROWS_PT, 16), jnp.float32),
            pltpu.VMEM((K, 16), jnp.float32),
            pltpu.VMEM_SHARED((NACC, 16), jnp.float32),
            pltpu.SemaphoreType.DMA,
        ],
        compiler_params=params,
    )
    agg_call = pl.kernel(
        _agg_body,
        out_type=jax.ShapeDtypeStruct((NC, NACC, H), jnp.float32),
        mesh=mesh,
        scratch_types=(
            [pltpu.VMEM((NCH, K), jnp.int32),
             pltpu.VMEM((NCH, K), jnp.int32)]
            + [pltpu.VMEM((K, H), jnp.float32)] * NBUF
            + [pltpu.VMEM_SHARED((NACC, H), jnp.float32)]
            + [pltpu.SemaphoreType.DMA] * (2 * NBUF)
        ),
        compiler_params=params,
    )
    return deg_call, agg_call


def _dinv(deg_ref):
    return lax.rsqrt(deg_ref[0, :, 0:1] + deg_ref[1, :, 0:1] + 1.0)


def _prep_kernel(deg_ref, x_ref, y_ref):
    dinv = _dinv(deg_ref)
    y_ref[0, :, :] = x_ref[:, 0:H] * dinv
    y_ref[1, :, :] = x_ref[:, H:IN_C] * dinv


def _mm_kernel(z_ref, y_ref, deg_ref, W1_ref, b1_ref, W2_ref, y2_ref):
    dinv = _dinv(deg_ref)
    s0 = (z_ref[0, :, :] + y_ref[0, :, :]) * dinv
    s1 = (z_ref[1, :, :] + y_ref[1, :, :]) * dinv
    h = (jnp.dot(s0, W1_ref[0:H, :], preferred_element_type=jnp.float32)
         + jnp.dot(s1, W1_ref[H:IN_C, :], preferred_element_type=jnp.float32)
         + b1_ref[...])
    h = jnp.maximum(h, 0.0)
    y2_ref[0, :, :] = jnp.dot(h, W2_ref[:, 0:H],
                              preferred_element_type=jnp.float32) * dinv
    y2_ref[1, :, :] = jnp.dot(h, W2_ref[:, H:OUT_C],
                              preferred_element_type=jnp.float32) * dinv


def _fin_kernel(z_ref, y_ref, deg_ref, b2_ref, out_ref):
    dinv = _dinv(deg_ref)
    out_ref[:, 0:H] = (z_ref[0, :, :] + y_ref[0, :, :]) * dinv + b2_ref[:, 0:H]
    out_ref[:, H:OUT_C] = (z_ref[1, :, :] + y_ref[1, :, :]) * dinv + b2_ref[:, H:OUT_C]


def _row_spec(w):
    return pl.BlockSpec((R, w), lambda i: (i, 0))


def _half_spec(w=None):
    return pl.BlockSpec((NC, R, w or H), lambda i: (0, i, 0))


def _full_spec(h, w):
    return pl.BlockSpec((h, w), lambda i: (0, 0))


@functools.lru_cache(maxsize=None)
def _tc_calls():
    prep = pl.pallas_call(
        _prep_kernel,
        grid=(GRID,),
        in_specs=[_half_spec(16), _row_spec(IN_C)],
        out_specs=_half_spec(),
        out_shape=jax.ShapeDtypeStruct((NC, N, H), jnp.float32),
    )
    mm = pl.pallas_call(
        _mm_kernel,
        grid=(GRID,),
        in_specs=[_half_spec(), _half_spec(), _half_spec(16),
                  _full_spec(IN_C, HID), _full_spec(1, HID),
                  _full_spec(HID, OUT_C)],
        out_specs=_half_spec(),
        out_shape=jax.ShapeDtypeStruct((NC, N, H), jnp.float32),
    )
    fin = pl.pallas_call(
        _fin_kernel,
        grid=(GRID,),
        in_specs=[_half_spec(), _half_spec(), _half_spec(16),
                  _full_spec(1, OUT_C)],
        out_specs=_row_spec(OUT_C),
        out_shape=jax.ShapeDtypeStruct((N, OUT_C), jnp.float32),
    )
    return prep, mm, fin


def kernel(x, edge_index, W1, b1, W2, b2):
    deg_call, agg_call = _sc_calls()
    prep, mm, fin = _tc_calls()

    src = edge_index[0]
    dst = edge_index[1]
    dst_deg = dst.reshape(NW, NCH_DEG, K)
    src_agg = jnp.stack([src, src + N]).reshape(NC, NS, NCH, K)
    dst_agg = dst.reshape(NS, NCH, K)

    ones16 = jnp.ones((K, 16), jnp.float32)
    zeros16 = jnp.zeros((ROWS_PT, 16), jnp.float32)
    zerosK = jnp.zeros((K, H), jnp.float32)

    degp = deg_call(dst_deg, zeros16, ones16)

    y1 = prep(degp, x)
    z1 = agg_call(y1.reshape(NC * N, H), src_agg, dst_agg, zerosK)
    y2 = mm(z1, y1, degp, W1, b1.reshape(1, HID), W2)
    z2 = agg_call(y2.reshape(NC * N, H), src_agg, dst_agg, zerosK)
    return fin(z2, y2, degp, b2.reshape(1, OUT_C))
